# baseline (device time: 10318 ns/iter reference)
import jax
import jax.numpy as jnp
from jax import lax
from jax.experimental import pallas as pl
from jax.experimental.pallas import tpu as pltpu

N_DEV = 4
N_TOK = 256
D_IN = 128
D_OUT = 256
E_LOCAL = 2
E_TOTAL = 8
ROWS = N_TOK // N_DEV


def kernel(x, router_W, route_idx, expert_W, shared_W):
    def body(x_ref, rw_ref, idx_ref, ew_ref, sw_ref, out_ref,
             partial_ref, recv_buf, send_sems, recv_sems):
        my_pos = lax.axis_index("i")

        barrier_sem = pltpu.get_barrier_semaphore()
        for k in range(1, N_DEV):
            pl.semaphore_signal(
                barrier_sem, inc=1,
                device_id=((my_pos + k) % N_DEV,),
                device_id_type=pl.DeviceIdType.MESH,
            )
        pl.semaphore_wait(barrier_sem, N_DEV - 1)

        xv = x_ref[:, :]
        scores = jnp.dot(xv, rw_ref[:, :], preferred_element_type=jnp.float32)
        s_max = jnp.max(scores, axis=-1, keepdims=True)
        e = jnp.exp(scores - s_max)
        probs = e / jnp.sum(e, axis=-1, keepdims=True)

        e_idx = idx_ref[:, :]
        iota = lax.broadcasted_iota(jnp.int32, (N_TOK, E_TOTAL), 1)
        onehot = iota == e_idx
        prob_sel = jnp.sum(jnp.where(onehot, probs, 0.0), axis=-1,
                           keepdims=True)

        contrib = jnp.zeros((N_TOK, D_OUT), jnp.float32)
        for j in range(E_LOCAL):
            ge = E_LOCAL * my_pos + j
            yj = jnp.dot(xv, ew_ref[j, :, :],
                         preferred_element_type=jnp.float32)
            coef = jnp.where(e_idx == ge, prob_sel, 0.0)
            contrib = contrib + coef * yj
        partial_ref[:, :] = contrib

        rdmas = []
        for k in range(1, N_DEV):
            target = (my_pos + k) % N_DEV
            rdma = pltpu.make_async_remote_copy(
                src_ref=partial_ref.at[pl.ds(target * ROWS, ROWS), :],
                dst_ref=recv_buf.at[k - 1],
                send_sem=send_sems.at[k - 1],
                recv_sem=recv_sems.at[k - 1],
                device_id=(target,),
                device_id_type=pl.DeviceIdType.MESH,
            )
            rdma.start()
            rdmas.append(rdma)

        x_blk = x_ref[pl.ds(my_pos * ROWS, ROWS), :]
        shared_blk = jnp.dot(x_blk, sw_ref[:, :],
                             preferred_element_type=jnp.float32)
        acc = shared_blk + partial_ref[pl.ds(my_pos * ROWS, ROWS), :]

        for rdma in rdmas:
            rdma.wait()
        acc = acc + recv_buf[0] + recv_buf[1] + recv_buf[2]
        out_ref[:, :] = acc

    return pl.pallas_call(
        body,
        out_shape=jax.ShapeDtypeStruct((ROWS, D_OUT), jnp.float32),
        in_specs=[
            pl.BlockSpec(memory_space=pltpu.VMEM),
            pl.BlockSpec(memory_space=pltpu.VMEM),
            pl.BlockSpec(memory_space=pltpu.VMEM),
            pl.BlockSpec(memory_space=pltpu.VMEM),
            pl.BlockSpec(memory_space=pltpu.VMEM),
        ],
        out_specs=pl.BlockSpec(memory_space=pltpu.VMEM),
        scratch_shapes=[
            pltpu.VMEM((N_TOK, D_OUT), jnp.float32),
            pltpu.VMEM((N_DEV - 1, ROWS, D_OUT), jnp.float32),
            pltpu.SemaphoreType.DMA((N_DEV - 1,)),
            pltpu.SemaphoreType.DMA((N_DEV - 1,)),
        ],
        compiler_params=pltpu.CompilerParams(collective_id=0),
    )(x, router_W, route_idx, expert_W, shared_W)
